# Initial kernel scaffold; baseline (speedup 1.0000x reference)
import jax
import jax.numpy as jnp
from jax.experimental import pallas as pl


def kernel(x, edge_index, edge_attr, batch, emb_W, emb_b, lin1_W, lin1_b, bn_g, bn_b, ln_g, ln_b, fc1_W, fc1_b, fcs_W, fcs_b, out_W, out_b):
    raise NotImplementedError("write your pallas kernel here")


# speedup vs baseline: 1.9937x; 1.9937x over previous
import functools

import jax
import jax.numpy as jnp
from jax import lax
from jax.experimental import pallas as pl
from jax.experimental.pallas import tpu as pltpu
from jax.experimental.pallas import tpu_sc as plsc

N = 10000
E = 320000
DX = 128
DE = 16
H = 64
H2 = 2 * H
FC = 128
LAYERS = 6
G = 64
EPS = 1e-5

F32 = jnp.float32

BN_ROWS = 1000
NB_N = N // BN_ROWS
BE_ROWS = 2000
NB_E = E // BE_ROWS

NC = 2
NS = 16
NW = NC * NS
CHUNK = 128
NCH = E // CHUNK
CH_BASE = NCH // NW
CH_REM = NCH % NW
STRIPE = 632
N_PAD = NS * STRIPE


def _emb_body(x_ref, w_ref, b_ref, o_ref):
    o_ref[...] = (
        jnp.dot(x_ref[...], w_ref[...], preferred_element_type=F32) + b_ref[...]
    )


def _emb(x, w, b2):
    return pl.pallas_call(
        _emb_body,
        grid=(NB_N,),
        in_specs=[
            pl.BlockSpec((BN_ROWS, DX), lambda i: (i, 0)),
            pl.BlockSpec((DX, H), lambda i: (0, 0)),
            pl.BlockSpec((1, H), lambda i: (0, 0)),
        ],
        out_specs=pl.BlockSpec((BN_ROWS, H), lambda i: (i, 0)),
        out_shape=jax.ShapeDtypeStruct((N, H), F32),
    )(x, w, b2)


def _pre_body(h_ref, wi_ref, wj_ref, a_ref, b_ref):
    h = h_ref[...]
    a_ref[...] = jnp.dot(h, wi_ref[...], preferred_element_type=F32)
    b_ref[...] = jnp.dot(h, wj_ref[...], preferred_element_type=F32)


def _pre(h, wi, wj):
    return pl.pallas_call(
        _pre_body,
        grid=(NB_N,),
        in_specs=[
            pl.BlockSpec((BN_ROWS, H), lambda i: (i, 0)),
            pl.BlockSpec((H, H2), lambda i: (0, 0)),
            pl.BlockSpec((H, H2), lambda i: (0, 0)),
        ],
        out_specs=[
            pl.BlockSpec((BN_ROWS, H2), lambda i: (i, 0)),
            pl.BlockSpec((BN_ROWS, H2), lambda i: (i, 0)),
        ],
        out_shape=[
            jax.ShapeDtypeStruct((N, H2), F32),
            jax.ShapeDtypeStruct((N, H2), F32),
        ],
    )(h, wi, wj)


def _stats_body(z_ref, attr_ref, we_ref, b_ref, s1_ref, s2_ref):
    i = pl.program_id(0)

    @pl.when(i == 0)
    def _():
        s1_ref[...] = jnp.zeros_like(s1_ref)
        s2_ref[...] = jnp.zeros_like(s2_ref)

    z = (
        z_ref[...]
        + jnp.dot(attr_ref[...], we_ref[...], preferred_element_type=F32)
        + b_ref[...]
    )
    s1_ref[...] = s1_ref[...] + jnp.sum(z, axis=0, keepdims=True)
    s2_ref[...] = s2_ref[...] + jnp.sum(z * z, axis=0, keepdims=True)


def _stats(z, attr, we, b2):
    return pl.pallas_call(
        _stats_body,
        grid=(NB_E,),
        in_specs=[
            pl.BlockSpec((BE_ROWS, H2), lambda i: (i, 0)),
            pl.BlockSpec((BE_ROWS, DE), lambda i: (i, 0)),
            pl.BlockSpec((DE, H2), lambda i: (0, 0)),
            pl.BlockSpec((1, H2), lambda i: (0, 0)),
        ],
        out_specs=[
            pl.BlockSpec((8, H2), lambda i: (0, 0)),
            pl.BlockSpec((8, H2), lambda i: (0, 0)),
        ],
        out_shape=[
            jax.ShapeDtypeStruct((8, H2), F32),
            jax.ShapeDtypeStruct((8, H2), F32),
        ],
    )(z, attr, we, b2)


def _main_body(z_ref, attr_ref, we_ref, b_ref, s1_ref, s2_ref, bng_ref, bnb_ref,
               msg_ref):
    z = (
        z_ref[...]
        + jnp.dot(attr_ref[...], we_ref[...], preferred_element_type=F32)
        + b_ref[...]
    )
    mu = s1_ref[0:1, :] * (1.0 / E)
    var = s2_ref[0:1, :] * (1.0 / E) - mu * mu
    scale = bng_ref[...] / jnp.sqrt(var + EPS)
    shift = bnb_ref[...] - mu * scale
    zn = z * scale + shift
    z1 = zn[:, :H]
    z2 = zn[:, H:]
    gated = jax.nn.sigmoid(z1) * jax.nn.softplus(z2)
    msg_ref[...] = jnp.concatenate([gated, jnp.zeros_like(gated)], axis=1)


def _gate(z, attr, we, b2, s1, s2, bng, bnb):
    return pl.pallas_call(
        _main_body,
        grid=(NB_E,),
        in_specs=[
            pl.BlockSpec((BE_ROWS, H2), lambda i: (i, 0)),
            pl.BlockSpec((BE_ROWS, DE), lambda i: (i, 0)),
            pl.BlockSpec((DE, H2), lambda i: (0, 0)),
            pl.BlockSpec((1, H2), lambda i: (0, 0)),
            pl.BlockSpec((8, H2), lambda i: (0, 0)),
            pl.BlockSpec((8, H2), lambda i: (0, 0)),
            pl.BlockSpec((1, H2), lambda i: (0, 0)),
            pl.BlockSpec((1, H2), lambda i: (0, 0)),
        ],
        out_specs=pl.BlockSpec((BE_ROWS, H2), lambda i: (i, 0)),
        out_shape=jax.ShapeDtypeStruct((E, H2), F32),
    )(z, attr, we, b2, s1, s2, bng, bnb)


def _post_body(p_ref, h_ref, g_ref, b_ref, o_ref):
    agg = p_ref[0] + p_ref[1]
    m = jnp.mean(agg, axis=-1, keepdims=True)
    d = agg - m
    v = jnp.mean(d * d, axis=-1, keepdims=True)
    ln = d / jnp.sqrt(v + EPS) * g_ref[...] + b_ref[...]
    o_ref[...] = jax.nn.softplus(ln + h_ref[...])


def _post(parts, h, g2, b2):
    return pl.pallas_call(
        _post_body,
        grid=(NB_N,),
        in_specs=[
            pl.BlockSpec((2, BN_ROWS, H), lambda i: (0, i, 0)),
            pl.BlockSpec((BN_ROWS, H), lambda i: (i, 0)),
            pl.BlockSpec((1, H), lambda i: (0, 0)),
            pl.BlockSpec((1, H), lambda i: (0, 0)),
        ],
        out_specs=pl.BlockSpec((BN_ROWS, H), lambda i: (i, 0)),
        out_shape=jax.ShapeDtypeStruct((N, H), F32),
    )(parts, h, g2, b2)


def _pool_body(bf_ref, h_ref, s_ref, c_ref):
    i = pl.program_id(0)

    @pl.when(i == 0)
    def _():
        s_ref[...] = jnp.zeros_like(s_ref)
        c_ref[...] = jnp.zeros_like(c_ref)

    gids = lax.broadcasted_iota(jnp.int32, (1, G), 1).astype(F32)
    oh = (bf_ref[...] == gids).astype(F32)
    s_ref[...] = s_ref[...] + lax.dot_general(
        oh, h_ref[...], (((0,), (0,)), ((), ())), preferred_element_type=F32
    )
    ones = jnp.ones((BN_ROWS, 8), F32)
    c_ref[...] = c_ref[...] + lax.dot_general(
        oh, ones, (((0,), (0,)), ((), ())), preferred_element_type=F32
    )


def _pool(batch_f, h):
    return pl.pallas_call(
        _pool_body,
        grid=(NB_N,),
        in_specs=[
            pl.BlockSpec((BN_ROWS, 1), lambda i: (i, 0)),
            pl.BlockSpec((BN_ROWS, H), lambda i: (i, 0)),
        ],
        out_specs=[
            pl.BlockSpec((G, H), lambda i: (0, 0)),
            pl.BlockSpec((G, 8), lambda i: (0, 0)),
        ],
        out_shape=[
            jax.ShapeDtypeStruct((G, H), F32),
            jax.ShapeDtypeStruct((G, 8), F32),
        ],
    )(batch_f, h)


def _head_body(s_ref, c_ref, w1_ref, b1_ref, ws_ref, bs_ref, wo_ref, bo_ref,
               o_ref):
    cnt = jnp.maximum(c_ref[:, 0:1], 1.0)
    mol = s_ref[...] / cnt
    mol = jax.nn.softplus(
        jnp.dot(mol, w1_ref[...], preferred_element_type=F32) + b1_ref[...]
    )
    for i in range(3):
        mol = jax.nn.softplus(
            jnp.dot(mol, ws_ref[i], preferred_element_type=F32)
            + bs_ref[i : i + 1, :]
        )
    res = jnp.sum(mol * wo_ref[...], axis=1, keepdims=True) + bo_ref[0:1, 0:1]
    o_ref[...] = jnp.broadcast_to(res, (G, FC))


def _head(sums, cnt, w1, b1, ws, bs, wo_row, bo):
    return pl.pallas_call(
        _head_body,
        grid=(1,),
        in_specs=[
            pl.BlockSpec((G, H), lambda i: (0, 0)),
            pl.BlockSpec((G, 8), lambda i: (0, 0)),
            pl.BlockSpec((H, FC), lambda i: (0, 0)),
            pl.BlockSpec((1, FC), lambda i: (0, 0)),
            pl.BlockSpec((3, FC, FC), lambda i: (0, 0, 0)),
            pl.BlockSpec((3, FC), lambda i: (0, 0)),
            pl.BlockSpec((1, FC), lambda i: (0, 0)),
            pl.BlockSpec((1, FC), lambda i: (0, 0)),
        ],
        out_specs=pl.BlockSpec((G, FC), lambda i: (0, 0)),
        out_shape=jax.ShapeDtypeStruct((G, FC), F32),
    )(sums, cnt, w1, b1, ws, bs, wo_row, bo)


def _sc_mesh():
    return plsc.VectorSubcoreMesh(core_axis_name="c", subcore_axis_name="s")


def _sc_gather_add(a, b, dst, src):
    @functools.partial(
        pl.kernel,
        out_type=jax.ShapeDtypeStruct((E, H2), F32),
        mesh=_sc_mesh(),
        scratch_types=[
            pltpu.VMEM((CHUNK,), jnp.int32),
            pltpu.VMEM((CHUNK,), jnp.int32),
            pltpu.VMEM((CHUNK, H2), F32),
            pltpu.VMEM((CHUNK, H2), F32),
            pltpu.SemaphoreType.DMA,
            pltpu.SemaphoreType.DMA,
        ],
    )
    def k(a_hbm, b_hbm, dst_hbm, src_hbm, z_hbm, idxa, idxb, bufa, bufb, sema,
          semb):
        wid = lax.axis_index("s") * NC + lax.axis_index("c")
        nch = jnp.where(wid < CH_REM, CH_BASE + 1, CH_BASE)

        def body(i, carry):
            base = (wid + i * NW) * CHUNK
            pltpu.sync_copy(dst_hbm.at[pl.ds(base, CHUNK)], idxa)
            pltpu.sync_copy(src_hbm.at[pl.ds(base, CHUNK)], idxb)
            ca = pltpu.async_copy(a_hbm.at[idxa], bufa, sema)
            cb = pltpu.async_copy(b_hbm.at[idxb], bufb, semb)
            ca.wait()
            cb.wait()

            def row(r, c2):
                for j in range(H2 // 16):
                    sl = pl.ds(16 * j, 16)
                    bufa[r, sl] = bufa[r, sl] + bufb[r, sl]
                return c2

            lax.fori_loop(0, CHUNK, row, 0, unroll=True)
            pltpu.sync_copy(bufa, z_hbm.at[pl.ds(base, CHUNK)])
            return carry

        lax.fori_loop(0, nch, body, 0)

    return k(a, b, dst, src)


def _sc_scatter_add(msg, dst, zrows):
    @functools.partial(
        pl.kernel,
        out_type=jax.ShapeDtypeStruct((NC, N_PAD, H2), F32),
        mesh=_sc_mesh(),
        scratch_types=[
            pltpu.VMEM((CHUNK,), jnp.int32),
            pltpu.VMEM((CHUNK, H2), F32),
            pltpu.VMEM_SHARED((N_PAD, H2), F32),
        ],
    )
    def k(msg_hbm, dst_hbm, z_hbm, out_hbm, idxv, bufm, acc):
        cid = lax.axis_index("c")
        sid = lax.axis_index("s")
        wid = sid * NC + cid
        pltpu.sync_copy(z_hbm, acc.at[pl.ds(sid * STRIPE, STRIPE)])
        plsc.subcore_barrier()
        nch = jnp.where(wid < CH_REM, CH_BASE + 1, CH_BASE)

        def body(i, carry):
            base = (wid + i * NW) * CHUNK
            pltpu.sync_copy(dst_hbm.at[pl.ds(base, CHUNK)], idxv)
            pltpu.sync_copy(msg_hbm.at[pl.ds(base, CHUNK)], bufm)
            pltpu.sync_copy(bufm, acc.at[idxv], add=True)
            return carry

        lax.fori_loop(0, nch, body, 0)
        plsc.subcore_barrier()
        pltpu.sync_copy(
            acc.at[pl.ds(sid * STRIPE, STRIPE)],
            out_hbm.at[cid, pl.ds(sid * STRIPE, STRIPE)],
        )

    return k(msg, dst, zrows)


def kernel(x, edge_index, edge_attr, batch, emb_W, emb_b, lin1_W, lin1_b,
           bn_g, bn_b, ln_g, ln_b, fc1_W, fc1_b, fcs_W, fcs_b, out_W, out_b):
    dst = edge_index[1]
    src = edge_index[0]
    zrows = jnp.zeros((STRIPE, H2), F32)

    h = _emb(x, emb_W, emb_b.reshape(1, H))

    for l in range(LAYERS):
        w = lin1_W[l]
        wi = w[:H, :]
        wj = w[H : 2 * H, :]
        we = w[2 * H :, :]
        b2 = lin1_b[l].reshape(1, H2)
        a_n, b_n = _pre(h, wi, wj)
        z = _sc_gather_add(a_n, b_n, dst, src)
        s1, s2 = _stats(z, edge_attr, we, b2)
        msg = _gate(z, edge_attr, we, b2, s1, s2,
                    bn_g[l].reshape(1, H2), bn_b[l].reshape(1, H2))
        parts = _sc_scatter_add(msg, dst, zrows)[:, :N, :H]
        h = _post(parts, h, ln_g[l].reshape(1, H), ln_b[l].reshape(1, H))

    sums, cnt = _pool(batch.astype(F32).reshape(N, 1), h)
    pooled = _head(sums, cnt, fc1_W, fc1_b.reshape(1, FC), fcs_W, fcs_b,
                   out_W.reshape(1, FC), jnp.broadcast_to(out_b.reshape(1, 1), (1, FC)))
    return pooled[:, 0]
